# baseline (speedup 1.0000x reference)
import functools

import jax
import jax.numpy as jnp
from jax import lax
from jax.experimental import pallas as pl
from jax.experimental.pallas import tpu as pltpu
from jax.experimental.pallas import tpu_sc as plsc

N = 50000
E = 800000
H = 256

NC, NS = 2, 16
NW = NC * NS

ROWS = 5120
NP = 5
NPAD = 2 * NP * ROWS
ZR = ROWS // NS
ZRA = 328
ACCR = ZRA * NS
G = 128
STG = G + 16
EB = 2000

_HIGH = lax.Precision.HIGHEST


def _deg_body(d1_hbm, d2_hbm, o1_hbm, o2_hbm, acc1, acc2, blk1, blk2):
    c = lax.axis_index("c")
    s = lax.axis_index("s")
    w = s * NC + c
    lanes = lax.iota(jnp.int32, 16)
    zv = jnp.zeros((16,), jnp.float32)

    def zloop(i, _):
        acc1[pl.ds(i * 16, 16)] = zv
        acc2[pl.ds(i * 16, 16)] = zv
        return 0

    lax.fori_loop(0, NPAD // 16, zloop, 0)

    nblk = E // EB
    nb = jnp.where(w < nblk % NW, nblk // NW + 1, nblk // NW)
    ones = jnp.ones((16,), jnp.float32)

    def bloop(b, _):
        off = (w + b * NW) * EB
        pltpu.sync_copy(d1_hbm.at[pl.ds(off, EB)], blk1)
        pltpu.sync_copy(d2_hbm.at[pl.ds(off, EB)], blk2)

        def gloop(g, _):
            d1v = blk1[pl.ds(g * 16, 16)]
            d2v = blk2[pl.ds(g * 16, 16)]
            for l in range(16):
                onelane = lanes == l
                plsc.addupdate_scatter(acc1, [d1v], ones, mask=onelane)
                plsc.addupdate_scatter(acc2, [d2v], ones, mask=onelane)
            return 0

        lax.fori_loop(0, EB // 16, gloop, 0)
        return 0

    lax.fori_loop(0, nb, bloop, 0)
    pltpu.sync_copy(acc1, o1_hbm.at[pl.ds(w * NPAD, NPAD)])
    pltpu.sync_copy(acc2, o2_hbm.at[pl.ds(w * NPAD, NPAD)])


@functools.lru_cache(maxsize=None)
def _deg_kernel():
    mesh = plsc.VectorSubcoreMesh(core_axis_name="c", subcore_axis_name="s")
    return pl.kernel(
        _deg_body,
        out_type=(jax.ShapeDtypeStruct((NW * NPAD,), jnp.float32),
                  jax.ShapeDtypeStruct((NW * NPAD,), jnp.float32)),
        mesh=mesh,
        compiler_params=pltpu.CompilerParams(needs_layout_passes=False),
        scratch_types=[
            pltpu.VMEM((NPAD,), jnp.float32),
            pltpu.VMEM((NPAD,), jnp.float32),
            pltpu.VMEM((EB,), jnp.int32),
            pltpu.VMEM((EB,), jnp.int32),
        ],
    )


def _seg1_body(xs_hbm, s_hbm, d_hbm, o_hbm, xsv, acc, sblk, dblk):
    c = lax.axis_index("c")
    s = lax.axis_index("s")
    w = s * NC + c
    lanes = lax.iota(jnp.int32, 16)
    zv = jnp.zeros((16,), jnp.float32)

    def zloop(i, _):
        acc[pl.ds(i * 16, 16)] = zv
        return 0

    lax.fori_loop(0, NPAD // 16, zloop, 0)
    pltpu.sync_copy(xs_hbm, xsv)

    nblk = E // EB
    nb = jnp.where(w < nblk % NW, nblk // NW + 1, nblk // NW)

    def bloop(b, _):
        off = (w + b * NW) * EB
        pltpu.sync_copy(s_hbm.at[pl.ds(off, EB)], sblk)
        pltpu.sync_copy(d_hbm.at[pl.ds(off, EB)], dblk)

        def gloop(g, _):
            sv = sblk[pl.ds(g * 16, 16)]
            dv = dblk[pl.ds(g * 16, 16)]
            vals = plsc.load_gather(xsv, [sv])
            for l in range(16):
                plsc.addupdate_scatter(acc, [dv], vals, mask=lanes == l)
            return 0

        lax.fori_loop(0, EB // 16, gloop, 0)
        return 0

    lax.fori_loop(0, nb, bloop, 0)
    pltpu.sync_copy(acc, o_hbm.at[pl.ds(w * NPAD, NPAD)])


@functools.lru_cache(maxsize=None)
def _seg1_kernel():
    mesh = plsc.VectorSubcoreMesh(core_axis_name="c", subcore_axis_name="s")
    return pl.kernel(
        _seg1_body,
        out_type=jax.ShapeDtypeStruct((NW * NPAD,), jnp.float32),
        mesh=mesh,
        compiler_params=pltpu.CompilerParams(needs_layout_passes=False),
        scratch_types=[
            pltpu.VMEM((NPAD,), jnp.float32),
            pltpu.VMEM((NPAD,), jnp.float32),
            pltpu.VMEM((EB,), jnp.int32),
            pltpu.VMEM((EB,), jnp.int32),
        ],
    )


TR = 352
NPA = 5
OPAD = NPA * NW * TR
GA = 32
STGA = GA + 16
EBA = 4000


def _agg_body(zs_hbm, s_hbm, d_hbm, out_hbm,
              acc, sblk, dblk, sstg, dstg, sfire, dfire, rows, sem):
    c = lax.axis_index("c")
    s = lax.axis_index("s")
    w = s * NC + c
    lanes = lax.iota(jnp.int32, 16)
    zv = jnp.zeros((16,), jnp.float32)

    def fire_chunk():
        pltpu.async_copy(zs_hbm.at[sfire], rows, sem).wait()
        def arow(e, _):
            e16 = jnp.full((16,), e, jnp.int32)
            ld16 = plsc.load_gather(dfire, [e16])

            def acol(j, _):
                cols = j * 16 + lanes
                data = plsc.load_gather(rows, [e16, cols])
                plsc.addupdate_scatter(acc, [ld16, cols], data)
                return 0

            lax.fori_loop(0, H // 16, acol, 0)
            return 0

        lax.fori_loop(0, GA, arow, 0)

    for p in range(NPA):
        lo = (p * NW + w) * TR

        def zloop(r, _):
            def zcol(j, _):
                plsc.store_scatter(
                    acc, [jnp.full((16,), r, jnp.int32), j * 16 + lanes], zv)
                return 0
            lax.fori_loop(0, H // 16, zcol, 0)
            return 0

        lax.fori_loop(0, TR, zloop, 0)

        def gloop(g, cnt):
            sv = sblk[pl.ds(g * 16, 16)]
            dv = dblk[pl.ds(g * 16, 16)]
            ldv = dv - lo
            m = (ldv >= 0) & (ldv < TR)
            mi = jnp.where(m, 1, 0).astype(jnp.int32)
            pos = cnt + plsc.cumsum(mi) - 1
            plsc.store_scatter(sstg, [pos], sv, mask=m)
            plsc.store_scatter(dstg, [pos], ldv, mask=m)
            cnt = cnt + jnp.sum(mi)

            @pl.when(cnt >= GA)
            def _():
                for k in range(GA // 16):
                    sfire[pl.ds(k * 16, 16)] = sstg[pl.ds(k * 16, 16)]
                    dfire[pl.ds(k * 16, 16)] = dstg[pl.ds(k * 16, 16)]
                fire_chunk()
                sstg[pl.ds(0, 16)] = sstg[pl.ds(GA, 16)]
                dstg[pl.ds(0, 16)] = dstg[pl.ds(GA, 16)]

            return jnp.where(cnt >= GA, cnt - GA, cnt)

        def bloop(b, cnt):
            pltpu.sync_copy(s_hbm.at[pl.ds(b * EBA, EBA)], sblk)
            pltpu.sync_copy(d_hbm.at[pl.ds(b * EBA, EBA)], dblk)
            return lax.fori_loop(0, EBA // 16, gloop, cnt)

        cnt = lax.fori_loop(0, E // EBA, bloop, jnp.int32(0))

        @pl.when(cnt > 0)
        def _():
            for k in range(GA // 16):
                pos16 = k * 16 + lanes
                keep = pos16 < cnt
                sfire[pl.ds(k * 16, 16)] = jnp.where(
                    keep, sstg[pl.ds(k * 16, 16)], 0)
                dfire[pl.ds(k * 16, 16)] = jnp.where(
                    keep, dstg[pl.ds(k * 16, 16)], TR)
            fire_chunk()

        pltpu.sync_copy(acc.at[pl.ds(0, TR)], out_hbm.at[pl.ds(lo, TR)])


@functools.lru_cache(maxsize=None)
def _agg_kernel():
    mesh = plsc.VectorSubcoreMesh(core_axis_name="c", subcore_axis_name="s")
    return pl.kernel(
        _agg_body,
        out_type=jax.ShapeDtypeStruct((OPAD, H), jnp.float32),
        mesh=mesh,
        compiler_params=pltpu.CompilerParams(needs_layout_passes=False),
        scratch_types=[
            pltpu.VMEM((TR + 8, H), jnp.float32),
            pltpu.VMEM((EBA,), jnp.int32),
            pltpu.VMEM((EBA,), jnp.int32),
            pltpu.VMEM((STGA,), jnp.int32),
            pltpu.VMEM((STGA,), jnp.int32),
            pltpu.VMEM((GA,), jnp.int32),
            pltpu.VMEM((GA,), jnp.int32),
            pltpu.VMEM((GA, H), jnp.float32),
            pltpu.SemaphoreType.DMA,
        ],
    )


_BLK = 1000


def _lin_stats_body(t_ref, w_ref, b_ref, h_ref, s1_ref, s2_ref):
    i = pl.program_id(0)
    h = jnp.maximum(
        lax.dot_general(t_ref[...], w_ref[...], (((1,), (0,)), ((), ())),
                        precision=_HIGH) + b_ref[...], 0.0)
    h_ref[...] = h

    @pl.when(i == 0)
    def _():
        s1_ref[...] = jnp.zeros_like(s1_ref)
        s2_ref[...] = jnp.zeros_like(s2_ref)

    ho = h.shape[1]
    s1_ref[...] += jnp.broadcast_to(jnp.sum(h, axis=0, keepdims=True),
                                    (8, ho))
    s2_ref[...] += jnp.broadcast_to(jnp.sum(h * h, axis=0, keepdims=True),
                                    (8, ho))


def _lin_stats(t, W, b):
    grid = N // _BLK
    ho = W.shape[1]
    ki = t.shape[1]
    return pl.pallas_call(
        _lin_stats_body,
        grid=(grid,),
        in_specs=[
            pl.BlockSpec((_BLK, ki), lambda i: (i, 0)),
            pl.BlockSpec(W.shape, lambda i: (0, 0)),
            pl.BlockSpec((ho,), lambda i: (0,)),
        ],
        out_specs=[
            pl.BlockSpec((_BLK, ho), lambda i: (i, 0)),
            pl.BlockSpec((8, ho), lambda i: (0, 0)),
            pl.BlockSpec((8, ho), lambda i: (0, 0)),
        ],
        out_shape=[
            jax.ShapeDtypeStruct((N, ho), jnp.float32),
            jax.ShapeDtypeStruct((8, ho), jnp.float32),
            jax.ShapeDtypeStruct((8, ho), jnp.float32),
        ],
    )(t, W, b)


def _bn_apply_body(h_ref, a_ref, c_ref, dis_ref, z_ref, zs_ref):
    z = h_ref[...] * a_ref[...] + c_ref[...]
    z_ref[...] = z
    zs_ref[...] = z * dis_ref[...]


def _bn_apply(h, alpha, beta, disc):
    grid = N // _BLK
    return pl.pallas_call(
        _bn_apply_body,
        grid=(grid,),
        in_specs=[
            pl.BlockSpec((_BLK, H), lambda i: (i, 0)),
            pl.BlockSpec((H,), lambda i: (0,)),
            pl.BlockSpec((H,), lambda i: (0,)),
            pl.BlockSpec((_BLK, 1), lambda i: (i, 0)),
        ],
        out_specs=[
            pl.BlockSpec((_BLK, H), lambda i: (i, 0)),
            pl.BlockSpec((_BLK, H), lambda i: (i, 0)),
        ],
        out_shape=[
            jax.ShapeDtypeStruct((N, H), jnp.float32),
            jax.ShapeDtypeStruct((N, H), jnp.float32),
        ],
    )(h, alpha, beta, disc)


def _combine_body(agg_ref, z_ref, dis_ref, dis2_ref, t_ref):
    t_ref[...] = agg_ref[...] * dis_ref[...] + z_ref[...] * dis2_ref[...]


def _combine(agg, z, disc, dis2c):
    grid = N // _BLK
    return pl.pallas_call(
        _combine_body,
        grid=(grid,),
        in_specs=[
            pl.BlockSpec((_BLK, H), lambda i: (i, 0)),
            pl.BlockSpec((_BLK, H), lambda i: (i, 0)),
            pl.BlockSpec((_BLK, 1), lambda i: (i, 0)),
            pl.BlockSpec((_BLK, 1), lambda i: (i, 0)),
        ],
        out_specs=pl.BlockSpec((_BLK, H), lambda i: (i, 0)),
        out_shape=jax.ShapeDtypeStruct((N, H), jnp.float32),
    )(agg, z, disc, dis2c)


def _mlp_body(z_ref, w1_ref, b1_ref, w2_ref, b2_ref, w3_ref, b3_ref, out_ref):
    dn = (((1,), (0,)), ((), ()))
    h = jnp.maximum(
        lax.dot_general(z_ref[...], w1_ref[...], dn, precision=_HIGH)
        + b1_ref[...], 0.0)
    h = jnp.maximum(
        lax.dot_general(h, w2_ref[...], dn, precision=_HIGH)
        + b2_ref[...], 0.0)
    out_ref[...] = lax.dot_general(h, w3_ref[...], dn,
                                   precision=_HIGH) + b3_ref[...]


def _projector(z, Wp1, bp1, Wp2, bp2, Wp3, bp3):
    grid = N // _BLK
    return pl.pallas_call(
        _mlp_body,
        grid=(grid,),
        in_specs=[
            pl.BlockSpec((_BLK, H), lambda i: (i, 0)),
            pl.BlockSpec((H, 512), lambda i: (0, 0)),
            pl.BlockSpec((512,), lambda i: (0,)),
            pl.BlockSpec((512, 256), lambda i: (0, 0)),
            pl.BlockSpec((256,), lambda i: (0,)),
            pl.BlockSpec((256, 256), lambda i: (0, 0)),
            pl.BlockSpec((256,), lambda i: (0,)),
        ],
        out_specs=pl.BlockSpec((_BLK, 256), lambda i: (i, 0)),
        out_shape=jax.ShapeDtypeStruct((N, 256), jnp.float32),
    )(z, Wp1, bp1, Wp2, bp2, Wp3, bp3)


_EPS = 1e-5


def _stats_to_affine(s1, s2, g, be):
    m = s1[0] / N
    v = s2[0] / N - m * m
    rstd = lax.rsqrt(jnp.maximum(v, 0.0) + _EPS)
    alpha = rstd * g
    return alpha, be - m * alpha


def kernel(x1, edge_index1, x2, edge_index2, W1, b1, W2, b2, W3, b3,
           g1, be1, g2, be2, g3, be3, Wp1, bp1, Wp2, bp2, Wp3, bp3):
    s1e, d1e = edge_index1[0], edge_index1[1]
    s2e, d2e = edge_index2[0], edge_index2[1]

    degp1, degp2 = _deg_kernel()(d1e, d2e)
    deg1 = jnp.sum(degp1.reshape(NW, NPAD), axis=0)[:N] + 1.0
    deg2 = jnp.sum(degp2.reshape(NW, NPAD), axis=0)[:N] + 1.0

    def enc(x, se, de, deg, W1_, b1_, params):
        (g1_, be1_, W2_, b2_, g2_, be2_, W3_, b3_, g3_, be3_) = params
        dis = lax.rsqrt(deg)
        dis2 = dis * dis
        x0 = x[:, 0]
        xs = jnp.zeros((NPAD,), jnp.float32).at[:N].set(x0 * dis)
        cpart = _seg1_kernel()(xs, se, de)
        cval = jnp.sum(cpart.reshape(NW, NPAD), axis=0)[:N] * dis + dis2 * x0
        disc = dis[:, None]
        dis2c = dis2[:, None]

        h, s1, s2 = _lin_stats(cval[:, None], W1_, b1_)
        alpha, beta = _stats_to_affine(s1, s2, g1_, be1_)
        z, zs = _bn_apply(h, alpha, beta, disc)

        agg = _agg_kernel()(zs, se, de)[:N]
        t = _combine(agg, z, disc, dis2c)
        h, s1, s2 = _lin_stats(t, W2_, b2_)
        alpha, beta = _stats_to_affine(s1, s2, g2_, be2_)
        z, zs = _bn_apply(h, alpha, beta, disc)

        agg = _agg_kernel()(zs, se, de)[:N]
        t = _combine(agg, z, disc, dis2c)
        h, s1, s2 = _lin_stats(t, W3_, b3_)
        alpha, beta = _stats_to_affine(s1, s2, g3_, be3_)
        z, _ = _bn_apply(h, alpha, beta, disc)
        return z

    params = (g1, be1, W2, b2, g2, be2, W3, b3, g3, be3)
    z1 = enc(x1, s1e, d1e, deg1, W1, b1, params)
    z2 = enc(x2, s2e, d2e, deg2, W1, b1, params)
    p1 = _projector(z1, Wp1, bp1, Wp2, bp2, Wp3, bp3)
    p2 = _projector(z2, Wp1, bp1, Wp2, bp2, Wp3, bp3)
    return (z1, z2, p1, p2)


# speedup vs baseline: 1.0471x; 1.0471x over previous
import functools

import jax
import jax.numpy as jnp
from jax import lax
from jax.experimental import pallas as pl
from jax.experimental.pallas import tpu as pltpu
from jax.experimental.pallas import tpu_sc as plsc

N = 50000
E = 800000
H = 256

NC, NS = 2, 16
NW = NC * NS

ROWS = 5120
NP = 5
NPAD = 2 * NP * ROWS
ZR = ROWS // NS
ZRA = 328
ACCR = ZRA * NS
G = 128
STG = G + 16
EB = 2000

_HIGH = lax.Precision.HIGHEST


def _deg_body(d1_hbm, d2_hbm, o1_hbm, o2_hbm, acc1, acc2, blk1, blk2):
    c = lax.axis_index("c")
    s = lax.axis_index("s")
    w = s * NC + c
    lanes = lax.iota(jnp.int32, 16)
    zv = jnp.zeros((16,), jnp.float32)

    def zloop(i, _):
        acc1[pl.ds(i * 16, 16)] = zv
        acc2[pl.ds(i * 16, 16)] = zv
        return 0

    lax.fori_loop(0, NPAD // 16, zloop, 0)

    nblk = E // EB
    nb = jnp.where(w < nblk % NW, nblk // NW + 1, nblk // NW)
    ones = jnp.ones((16,), jnp.float32)

    def bloop(b, _):
        off = (w + b * NW) * EB
        pltpu.sync_copy(d1_hbm.at[pl.ds(off, EB)], blk1)
        pltpu.sync_copy(d2_hbm.at[pl.ds(off, EB)], blk2)

        def gloop(g, _):
            d1v = blk1[pl.ds(g * 16, 16)]
            d2v = blk2[pl.ds(g * 16, 16)]
            for l in range(16):
                onelane = lanes == l
                plsc.addupdate_scatter(acc1, [d1v], ones, mask=onelane)
                plsc.addupdate_scatter(acc2, [d2v], ones, mask=onelane)
            return 0

        lax.fori_loop(0, EB // 16, gloop, 0)
        return 0

    lax.fori_loop(0, nb, bloop, 0)
    pltpu.sync_copy(acc1, o1_hbm.at[pl.ds(w * NPAD, NPAD)])
    pltpu.sync_copy(acc2, o2_hbm.at[pl.ds(w * NPAD, NPAD)])


@functools.lru_cache(maxsize=None)
def _deg_kernel():
    mesh = plsc.VectorSubcoreMesh(core_axis_name="c", subcore_axis_name="s")
    return pl.kernel(
        _deg_body,
        out_type=(jax.ShapeDtypeStruct((NW * NPAD,), jnp.float32),
                  jax.ShapeDtypeStruct((NW * NPAD,), jnp.float32)),
        mesh=mesh,
        compiler_params=pltpu.CompilerParams(needs_layout_passes=False),
        scratch_types=[
            pltpu.VMEM((NPAD,), jnp.float32),
            pltpu.VMEM((NPAD,), jnp.float32),
            pltpu.VMEM((EB,), jnp.int32),
            pltpu.VMEM((EB,), jnp.int32),
        ],
    )


def _seg1_body(xs_hbm, s_hbm, d_hbm, o_hbm, xsv, acc, sblk, dblk):
    c = lax.axis_index("c")
    s = lax.axis_index("s")
    w = s * NC + c
    lanes = lax.iota(jnp.int32, 16)
    zv = jnp.zeros((16,), jnp.float32)

    def zloop(i, _):
        acc[pl.ds(i * 16, 16)] = zv
        return 0

    lax.fori_loop(0, NPAD // 16, zloop, 0)
    pltpu.sync_copy(xs_hbm, xsv)

    nblk = E // EB
    nb = jnp.where(w < nblk % NW, nblk // NW + 1, nblk // NW)

    def bloop(b, _):
        off = (w + b * NW) * EB
        pltpu.sync_copy(s_hbm.at[pl.ds(off, EB)], sblk)
        pltpu.sync_copy(d_hbm.at[pl.ds(off, EB)], dblk)

        def gloop(g, _):
            sv = sblk[pl.ds(g * 16, 16)]
            dv = dblk[pl.ds(g * 16, 16)]
            vals = plsc.load_gather(xsv, [sv])
            for l in range(16):
                plsc.addupdate_scatter(acc, [dv], vals, mask=lanes == l)
            return 0

        lax.fori_loop(0, EB // 16, gloop, 0)
        return 0

    lax.fori_loop(0, nb, bloop, 0)
    pltpu.sync_copy(acc, o_hbm.at[pl.ds(w * NPAD, NPAD)])


@functools.lru_cache(maxsize=None)
def _seg1_kernel():
    mesh = plsc.VectorSubcoreMesh(core_axis_name="c", subcore_axis_name="s")
    return pl.kernel(
        _seg1_body,
        out_type=jax.ShapeDtypeStruct((NW * NPAD,), jnp.float32),
        mesh=mesh,
        compiler_params=pltpu.CompilerParams(needs_layout_passes=False),
        scratch_types=[
            pltpu.VMEM((NPAD,), jnp.float32),
            pltpu.VMEM((NPAD,), jnp.float32),
            pltpu.VMEM((EB,), jnp.int32),
            pltpu.VMEM((EB,), jnp.int32),
        ],
    )


TR = 352
NPA = 5
OPAD = NPA * NW * TR
GA = 32
STGA = GA + 16
EBA = 4000


def _agg_body(zs_hbm, s_hbm, d_hbm, out_hbm,
              acc, sblk, dblk, sstg, dstg, sfire, dfire, rows, sem):
    c = lax.axis_index("c")
    s = lax.axis_index("s")
    w = s * NC + c
    lanes = lax.iota(jnp.int32, 16)
    zv = jnp.zeros((16,), jnp.float32)

    def fire_chunk():
        pltpu.async_copy(zs_hbm.at[sfire], rows, sem).wait()
        def arow(e, _):
            e16 = jnp.full((16,), e, jnp.int32)
            ld16 = plsc.load_gather(dfire, [e16])
            for j in range(H // 16):
                cols = j * 16 + lanes
                data = plsc.load_gather(rows, [e16, cols])
                plsc.addupdate_scatter(acc, [ld16, cols], data)
            return 0

        lax.fori_loop(0, GA, arow, 0)

    for p in range(NPA):
        lo = (p * NW + w) * TR

        def zloop(r, _):
            def zcol(j, _):
                plsc.store_scatter(
                    acc, [jnp.full((16,), r, jnp.int32), j * 16 + lanes], zv)
                return 0
            lax.fori_loop(0, H // 16, zcol, 0)
            return 0

        lax.fori_loop(0, TR, zloop, 0)

        def gloop(g, cnt):
            sv = sblk[pl.ds(g * 16, 16)]
            dv = dblk[pl.ds(g * 16, 16)]
            ldv = dv - lo
            m = (ldv >= 0) & (ldv < TR)
            mi = jnp.where(m, 1, 0).astype(jnp.int32)
            pos = cnt + plsc.cumsum(mi) - 1
            plsc.store_scatter(sstg, [pos], sv, mask=m)
            plsc.store_scatter(dstg, [pos], ldv, mask=m)
            cnt = cnt + jnp.sum(mi)

            @pl.when(cnt >= GA)
            def _():
                for k in range(GA // 16):
                    sfire[pl.ds(k * 16, 16)] = sstg[pl.ds(k * 16, 16)]
                    dfire[pl.ds(k * 16, 16)] = dstg[pl.ds(k * 16, 16)]
                fire_chunk()
                sstg[pl.ds(0, 16)] = sstg[pl.ds(GA, 16)]
                dstg[pl.ds(0, 16)] = dstg[pl.ds(GA, 16)]

            return jnp.where(cnt >= GA, cnt - GA, cnt)

        def bloop(b, cnt):
            pltpu.sync_copy(s_hbm.at[pl.ds(b * EBA, EBA)], sblk)
            pltpu.sync_copy(d_hbm.at[pl.ds(b * EBA, EBA)], dblk)
            return lax.fori_loop(0, EBA // 16, gloop, cnt)

        cnt = lax.fori_loop(0, E // EBA, bloop, jnp.int32(0))

        @pl.when(cnt > 0)
        def _():
            for k in range(GA // 16):
                pos16 = k * 16 + lanes
                keep = pos16 < cnt
                sfire[pl.ds(k * 16, 16)] = jnp.where(
                    keep, sstg[pl.ds(k * 16, 16)], 0)
                dfire[pl.ds(k * 16, 16)] = jnp.where(
                    keep, dstg[pl.ds(k * 16, 16)], TR)
            fire_chunk()

        pltpu.sync_copy(acc.at[pl.ds(0, TR)], out_hbm.at[pl.ds(lo, TR)])


@functools.lru_cache(maxsize=None)
def _agg_kernel():
    mesh = plsc.VectorSubcoreMesh(core_axis_name="c", subcore_axis_name="s")
    return pl.kernel(
        _agg_body,
        out_type=jax.ShapeDtypeStruct((OPAD, H), jnp.float32),
        mesh=mesh,
        compiler_params=pltpu.CompilerParams(needs_layout_passes=False),
        scratch_types=[
            pltpu.VMEM((TR + 8, H), jnp.float32),
            pltpu.VMEM((EBA,), jnp.int32),
            pltpu.VMEM((EBA,), jnp.int32),
            pltpu.VMEM((STGA,), jnp.int32),
            pltpu.VMEM((STGA,), jnp.int32),
            pltpu.VMEM((GA,), jnp.int32),
            pltpu.VMEM((GA,), jnp.int32),
            pltpu.VMEM((GA, H), jnp.float32),
            pltpu.SemaphoreType.DMA,
        ],
    )


_BLK = 1000


def _lin_stats_body(t_ref, w_ref, b_ref, h_ref, s1_ref, s2_ref):
    i = pl.program_id(0)
    h = jnp.maximum(
        lax.dot_general(t_ref[...], w_ref[...], (((1,), (0,)), ((), ())),
                        precision=_HIGH) + b_ref[...], 0.0)
    h_ref[...] = h

    @pl.when(i == 0)
    def _():
        s1_ref[...] = jnp.zeros_like(s1_ref)
        s2_ref[...] = jnp.zeros_like(s2_ref)

    ho = h.shape[1]
    s1_ref[...] += jnp.broadcast_to(jnp.sum(h, axis=0, keepdims=True),
                                    (8, ho))
    s2_ref[...] += jnp.broadcast_to(jnp.sum(h * h, axis=0, keepdims=True),
                                    (8, ho))


def _lin_stats(t, W, b):
    grid = N // _BLK
    ho = W.shape[1]
    ki = t.shape[1]
    return pl.pallas_call(
        _lin_stats_body,
        grid=(grid,),
        in_specs=[
            pl.BlockSpec((_BLK, ki), lambda i: (i, 0)),
            pl.BlockSpec(W.shape, lambda i: (0, 0)),
            pl.BlockSpec((ho,), lambda i: (0,)),
        ],
        out_specs=[
            pl.BlockSpec((_BLK, ho), lambda i: (i, 0)),
            pl.BlockSpec((8, ho), lambda i: (0, 0)),
            pl.BlockSpec((8, ho), lambda i: (0, 0)),
        ],
        out_shape=[
            jax.ShapeDtypeStruct((N, ho), jnp.float32),
            jax.ShapeDtypeStruct((8, ho), jnp.float32),
            jax.ShapeDtypeStruct((8, ho), jnp.float32),
        ],
    )(t, W, b)


def _bn_apply_body(h_ref, a_ref, c_ref, dis_ref, z_ref, zs_ref):
    z = h_ref[...] * a_ref[...] + c_ref[...]
    z_ref[...] = z
    zs_ref[...] = z * dis_ref[...]


def _bn_apply(h, alpha, beta, disc):
    grid = N // _BLK
    return pl.pallas_call(
        _bn_apply_body,
        grid=(grid,),
        in_specs=[
            pl.BlockSpec((_BLK, H), lambda i: (i, 0)),
            pl.BlockSpec((H,), lambda i: (0,)),
            pl.BlockSpec((H,), lambda i: (0,)),
            pl.BlockSpec((_BLK, 1), lambda i: (i, 0)),
        ],
        out_specs=[
            pl.BlockSpec((_BLK, H), lambda i: (i, 0)),
            pl.BlockSpec((_BLK, H), lambda i: (i, 0)),
        ],
        out_shape=[
            jax.ShapeDtypeStruct((N, H), jnp.float32),
            jax.ShapeDtypeStruct((N, H), jnp.float32),
        ],
    )(h, alpha, beta, disc)


def _combine_body(agg_ref, z_ref, dis_ref, dis2_ref, t_ref):
    t_ref[...] = agg_ref[...] * dis_ref[...] + z_ref[...] * dis2_ref[...]


def _combine(agg, z, disc, dis2c):
    grid = N // _BLK
    return pl.pallas_call(
        _combine_body,
        grid=(grid,),
        in_specs=[
            pl.BlockSpec((_BLK, H), lambda i: (i, 0)),
            pl.BlockSpec((_BLK, H), lambda i: (i, 0)),
            pl.BlockSpec((_BLK, 1), lambda i: (i, 0)),
            pl.BlockSpec((_BLK, 1), lambda i: (i, 0)),
        ],
        out_specs=pl.BlockSpec((_BLK, H), lambda i: (i, 0)),
        out_shape=jax.ShapeDtypeStruct((N, H), jnp.float32),
    )(agg, z, disc, dis2c)


def _mlp_body(z_ref, w1_ref, b1_ref, w2_ref, b2_ref, w3_ref, b3_ref, out_ref):
    dn = (((1,), (0,)), ((), ()))
    h = jnp.maximum(
        lax.dot_general(z_ref[...], w1_ref[...], dn, precision=_HIGH)
        + b1_ref[...], 0.0)
    h = jnp.maximum(
        lax.dot_general(h, w2_ref[...], dn, precision=_HIGH)
        + b2_ref[...], 0.0)
    out_ref[...] = lax.dot_general(h, w3_ref[...], dn,
                                   precision=_HIGH) + b3_ref[...]


def _projector(z, Wp1, bp1, Wp2, bp2, Wp3, bp3):
    grid = N // _BLK
    return pl.pallas_call(
        _mlp_body,
        grid=(grid,),
        in_specs=[
            pl.BlockSpec((_BLK, H), lambda i: (i, 0)),
            pl.BlockSpec((H, 512), lambda i: (0, 0)),
            pl.BlockSpec((512,), lambda i: (0,)),
            pl.BlockSpec((512, 256), lambda i: (0, 0)),
            pl.BlockSpec((256,), lambda i: (0,)),
            pl.BlockSpec((256, 256), lambda i: (0, 0)),
            pl.BlockSpec((256,), lambda i: (0,)),
        ],
        out_specs=pl.BlockSpec((_BLK, 256), lambda i: (i, 0)),
        out_shape=jax.ShapeDtypeStruct((N, 256), jnp.float32),
    )(z, Wp1, bp1, Wp2, bp2, Wp3, bp3)


_EPS = 1e-5


def _stats_to_affine(s1, s2, g, be):
    m = s1[0] / N
    v = s2[0] / N - m * m
    rstd = lax.rsqrt(jnp.maximum(v, 0.0) + _EPS)
    alpha = rstd * g
    return alpha, be - m * alpha


def kernel(x1, edge_index1, x2, edge_index2, W1, b1, W2, b2, W3, b3,
           g1, be1, g2, be2, g3, be3, Wp1, bp1, Wp2, bp2, Wp3, bp3):
    s1e, d1e = edge_index1[0], edge_index1[1]
    s2e, d2e = edge_index2[0], edge_index2[1]

    degp1, degp2 = _deg_kernel()(d1e, d2e)
    deg1 = jnp.sum(degp1.reshape(NW, NPAD), axis=0)[:N] + 1.0
    deg2 = jnp.sum(degp2.reshape(NW, NPAD), axis=0)[:N] + 1.0

    def enc(x, se, de, deg, W1_, b1_, params):
        (g1_, be1_, W2_, b2_, g2_, be2_, W3_, b3_, g3_, be3_) = params
        dis = lax.rsqrt(deg)
        dis2 = dis * dis
        x0 = x[:, 0]
        xs = jnp.zeros((NPAD,), jnp.float32).at[:N].set(x0 * dis)
        cpart = _seg1_kernel()(xs, se, de)
        cval = jnp.sum(cpart.reshape(NW, NPAD), axis=0)[:N] * dis + dis2 * x0
        disc = dis[:, None]
        dis2c = dis2[:, None]

        h, s1, s2 = _lin_stats(cval[:, None], W1_, b1_)
        alpha, beta = _stats_to_affine(s1, s2, g1_, be1_)
        z, zs = _bn_apply(h, alpha, beta, disc)

        agg = _agg_kernel()(zs, se, de)[:N]
        t = _combine(agg, z, disc, dis2c)
        h, s1, s2 = _lin_stats(t, W2_, b2_)
        alpha, beta = _stats_to_affine(s1, s2, g2_, be2_)
        z, zs = _bn_apply(h, alpha, beta, disc)

        agg = _agg_kernel()(zs, se, de)[:N]
        t = _combine(agg, z, disc, dis2c)
        h, s1, s2 = _lin_stats(t, W3_, b3_)
        alpha, beta = _stats_to_affine(s1, s2, g3_, be3_)
        z, _ = _bn_apply(h, alpha, beta, disc)
        return z

    params = (g1, be1, W2, b2, g2, be2, W3, b3, g3, be3)
    z1 = enc(x1, s1e, d1e, deg1, W1, b1, params)
    z2 = enc(x2, s2e, d2e, deg2, W1, b1, params)
    p1 = _projector(z1, Wp1, bp1, Wp2, bp2, Wp3, bp3)
    p2 = _projector(z2, Wp1, bp1, Wp2, bp2, Wp3, bp3)
    return (z1, z2, p1, p2)
